# Initial kernel scaffold; baseline (speedup 1.0000x reference)
import jax
import jax.numpy as jnp
from jax.experimental import pallas as pl


def kernel(x, edge_index, W1, b1, W2, b2):
    raise NotImplementedError("write your pallas kernel here")


# speedup vs baseline: 33.9068x; 33.9068x over previous
import functools

import jax
import jax.numpy as jnp
from jax import lax
from jax.experimental import pallas as pl
from jax.experimental.pallas import tpu as pltpu
from jax.experimental.pallas import tpu_sc as plsc

NN = 10000
EE = 320000
DD = 128
FF = 16
CC = 7

NC = 2
NS = 16
NW = NC * NS

IW = 128
KK = 8
N_OUTER = 10
EP = NW * N_OUTER * KK * IW
RI = EP // IW

NP = 10112
ZR = NP // NS

_mesh = plsc.VectorSubcoreMesh(core_axis_name="c", subcore_axis_name="s")
_sc_params = pltpu.CompilerParams(use_tc_tiling_on_sc=False)


@functools.partial(
    pl.kernel,
    out_type=jax.ShapeDtypeStruct((NC, NP, FF), jnp.float32),
    mesh=_mesh,
    compiler_params=_sc_params,
    scratch_types=[
        pltpu.VMEM((KK, IW), jnp.int32),
        pltpu.VMEM((IW, FF), jnp.float32),
        pltpu.VMEM((ZR, FF), jnp.float32),
        pltpu.VMEM_SHARED((NP, FF), jnp.float32),
    ],
)
def _deg_sc(dst_hbm, out_hbm, didx, ones, zbuf, acc):
    cid = lax.axis_index("c")
    sid = lax.axis_index("s")
    wid = cid * NS + sid

    @pl.loop(0, ZR)
    def _(i):
        zbuf[i, :] = jnp.zeros((FF,), jnp.float32)

    @pl.loop(0, IW)
    def _(i):
        ones[i, :] = jnp.ones((FF,), jnp.float32)

    pltpu.sync_copy(zbuf, acc.at[pl.ds(sid * ZR, ZR)])
    plsc.subcore_barrier()

    base = wid * (N_OUTER * KK)

    @pl.loop(0, N_OUTER)
    def _(it):
        r0 = base + it * KK
        pltpu.sync_copy(dst_hbm.at[pl.ds(r0, KK)], didx)
        for j in range(KK):
            pltpu.sync_copy(ones, acc.at[didx.at[j]], add=True)

    plsc.subcore_barrier()
    pltpu.sync_copy(acc.at[pl.ds(sid * ZR, ZR)],
                    out_hbm.at[cid, pl.ds(sid * ZR, ZR)])


@functools.partial(
    pl.kernel,
    out_type=jax.ShapeDtypeStruct((NC, NP, FF), jnp.float32),
    mesh=_mesh,
    compiler_params=_sc_params,
    scratch_types=[
        pltpu.VMEM((KK, IW), jnp.int32),
        pltpu.VMEM((KK, IW), jnp.int32),
        pltpu.VMEM((KK * IW, FF), jnp.float32),
        pltpu.VMEM((ZR, FF), jnp.float32),
        pltpu.VMEM_SHARED((NP, FF), jnp.float32),
        pltpu.SemaphoreType.DMA,
    ],
)
def _agg_sc(h_hbm, src_hbm, dst_hbm, out_hbm, sidx, didx, rows, zbuf, acc, gsem):
    cid = lax.axis_index("c")
    sid = lax.axis_index("s")
    wid = cid * NS + sid

    @pl.loop(0, ZR)
    def _(i):
        zbuf[i, :] = jnp.zeros((FF,), jnp.float32)

    pltpu.sync_copy(zbuf, acc.at[pl.ds(sid * ZR, ZR)])
    plsc.subcore_barrier()

    base = wid * (N_OUTER * KK)

    @pl.loop(0, N_OUTER)
    def _(it):
        r0 = base + it * KK
        pltpu.sync_copy(src_hbm.at[pl.ds(r0, KK)], sidx)
        pltpu.sync_copy(dst_hbm.at[pl.ds(r0, KK)], didx)
        copies = []
        for j in range(KK):
            copies.append(pltpu.async_copy(
                h_hbm.at[sidx.at[j]], rows.at[pl.ds(j * IW, IW)], gsem))
        for c in copies:
            c.wait()
        for j in range(KK):
            pltpu.sync_copy(rows.at[pl.ds(j * IW, IW)], acc.at[didx.at[j]],
                            add=True)

    plsc.subcore_barrier()
    pltpu.sync_copy(acc.at[pl.ds(sid * ZR, ZR)],
                    out_hbm.at[cid, pl.ds(sid * ZR, ZR)])


_BN = 1000
_GRID = NN // _BN


def _tc1_body(x_ref, w1_ref, degp_ref, h1s_ref, dis_ref):
    deg = degp_ref[0, :, :] + degp_ref[1, :, :] + 1.0
    dis = lax.rsqrt(deg)
    h1 = jnp.dot(x_ref[...], w1_ref[...], preferred_element_type=jnp.float32)
    h1s_ref[...] = h1 * dis
    dis_ref[...] = dis


def _tc1(x, W1, degp):
    return pl.pallas_call(
        _tc1_body,
        grid=(_GRID,),
        in_specs=[
            pl.BlockSpec((_BN, DD), lambda i: (i, 0)),
            pl.BlockSpec((DD, FF), lambda i: (0, 0)),
            pl.BlockSpec((NC, _BN, FF), lambda i: (0, i, 0)),
        ],
        out_specs=[
            pl.BlockSpec((_BN, FF), lambda i: (i, 0)),
            pl.BlockSpec((_BN, FF), lambda i: (i, 0)),
        ],
        out_shape=[
            jax.ShapeDtypeStruct((NN, FF), jnp.float32),
            jax.ShapeDtypeStruct((NN, FF), jnp.float32),
        ],
    )(x, W1, degp)


def _tc2_body(aggp_ref, h1s_ref, dis_ref, b1_ref, w2_ref, h2s_ref):
    agg = aggp_ref[0, :, :] + aggp_ref[1, :, :] + h1s_ref[...]
    z = jnp.maximum(dis_ref[...] * agg + b1_ref[...], 0.0)
    h2 = jnp.dot(z, w2_ref[...], preferred_element_type=jnp.float32)
    h2s_ref[...] = h2 * dis_ref[...]


def _tc2(aggp, h1s, dis, b1, W2p):
    return pl.pallas_call(
        _tc2_body,
        grid=(_GRID,),
        in_specs=[
            pl.BlockSpec((NC, _BN, FF), lambda i: (0, i, 0)),
            pl.BlockSpec((_BN, FF), lambda i: (i, 0)),
            pl.BlockSpec((_BN, FF), lambda i: (i, 0)),
            pl.BlockSpec((1, FF), lambda i: (0, 0)),
            pl.BlockSpec((FF, FF), lambda i: (0, 0)),
        ],
        out_specs=pl.BlockSpec((_BN, FF), lambda i: (i, 0)),
        out_shape=jax.ShapeDtypeStruct((NN, FF), jnp.float32),
    )(aggp, h1s, dis, b1, W2p)


def _tc3_body(aggp_ref, h2s_ref, dis_ref, b2_ref, out_ref):
    agg = aggp_ref[0, :, :] + aggp_ref[1, :, :] + h2s_ref[...]
    z = dis_ref[...] * agg + b2_ref[...]
    col = lax.broadcasted_iota(jnp.int32, (_BN, FF), 1)
    zm = jnp.where(col < CC, z, -jnp.inf)
    m = jnp.max(zm, axis=1, keepdims=True)
    lse = jnp.log(jnp.sum(jnp.exp(zm - m), axis=1, keepdims=True)) + m
    out_ref[...] = zm - lse


def _tc3(aggp, h2s, dis, b2):
    return pl.pallas_call(
        _tc3_body,
        grid=(_GRID,),
        in_specs=[
            pl.BlockSpec((NC, _BN, FF), lambda i: (0, i, 0)),
            pl.BlockSpec((_BN, FF), lambda i: (i, 0)),
            pl.BlockSpec((_BN, FF), lambda i: (i, 0)),
            pl.BlockSpec((1, FF), lambda i: (0, 0)),
        ],
        out_specs=pl.BlockSpec((_BN, FF), lambda i: (i, 0)),
        out_shape=jax.ShapeDtypeStruct((NN, FF), jnp.float32),
    )(aggp, h2s, dis, b2)


def kernel(x, edge_index, W1, b1, W2, b2):
    pad_n = EP - EE
    src_pad = jnp.zeros((pad_n,), jnp.int32)
    dst_pad = NN + (jnp.arange(pad_n, dtype=jnp.int32) % NS)
    src2d = jnp.concatenate([edge_index[0], src_pad]).reshape(RI, IW)
    dst2d = jnp.concatenate([edge_index[1], dst_pad]).reshape(RI, IW)

    W2p = jnp.zeros((FF, FF), jnp.float32).at[:, :CC].set(W2)
    b1r = b1.reshape(1, FF)
    b2r = jnp.zeros((1, FF), jnp.float32).at[0, :CC].set(b2)

    degp = _deg_sc(dst2d)
    h1s, dis = _tc1(x, W1, degp)
    agg1p = _agg_sc(h1s, src2d, dst2d)
    h2s = _tc2(agg1p, h1s, dis, b1r, W2p)
    agg2p = _agg_sc(h2s, src2d, dst2d)
    out16 = _tc3(agg2p, h2s, dis, b2r)
    return out16[:, :CC]


# speedup vs baseline: 38.7516x; 1.1429x over previous
import functools

import jax
import jax.numpy as jnp
from jax import lax
from jax.experimental import pallas as pl
from jax.experimental.pallas import tpu as pltpu
from jax.experimental.pallas import tpu_sc as plsc

NN = 10000
EE = 320000
DD = 128
FF = 16
CC = 7

NC = 2
NS = 16
NW = NC * NS

IW = 128
KK = 8
N_OUTER = 10
EP = NW * N_OUTER * KK * IW
RI = EP // IW

NP = 10112
ZR = NP // NS

_mesh = plsc.VectorSubcoreMesh(core_axis_name="c", subcore_axis_name="s")
_sc_params = pltpu.CompilerParams(use_tc_tiling_on_sc=False)


RW = N_OUTER * KK


@functools.partial(
    pl.kernel,
    out_type=jax.ShapeDtypeStruct((NC, NP, FF), jnp.float32),
    mesh=_mesh,
    compiler_params=_sc_params,
    scratch_types=[
        pltpu.VMEM((RW, IW), jnp.int32),
        pltpu.VMEM((IW, FF), jnp.float32),
        pltpu.VMEM((ZR, FF), jnp.float32),
        pltpu.VMEM_SHARED((NP, FF), jnp.float32),
        pltpu.SemaphoreType.DMA,
    ],
)
def _deg_sc(dst_hbm, out_hbm, didx, ones, zbuf, acc, ssem):
    cid = lax.axis_index("c")
    sid = lax.axis_index("s")
    wid = cid * NS + sid

    pltpu.sync_copy(dst_hbm.at[pl.ds(wid * RW, RW)], didx)

    @pl.loop(0, ZR)
    def _(i):
        zbuf[i, :] = jnp.zeros((FF,), jnp.float32)

    @pl.loop(0, IW)
    def _(i):
        ones[i, :] = jnp.ones((FF,), jnp.float32)

    pltpu.sync_copy(zbuf, acc.at[pl.ds(sid * ZR, ZR)])
    plsc.subcore_barrier()

    @pl.loop(0, N_OUTER // 2)
    def _(it):
        copies = []
        for j in range(2 * KK):
            copies.append(pltpu.async_copy(
                ones, acc.at[didx.at[it * 2 * KK + j]], ssem, add=True))
        for c in copies:
            c.wait()

    plsc.subcore_barrier()
    pltpu.sync_copy(acc.at[pl.ds(sid * ZR, ZR)],
                    out_hbm.at[cid, pl.ds(sid * ZR, ZR)])


NG = N_OUTER // 2


@functools.partial(
    pl.kernel,
    out_type=jax.ShapeDtypeStruct((NC, NP, FF), jnp.float32),
    mesh=_mesh,
    compiler_params=_sc_params,
    scratch_types=[
        pltpu.VMEM((RW, IW), jnp.int32),
        pltpu.VMEM((RW, IW), jnp.int32),
        pltpu.VMEM((KK * IW, FF), jnp.float32),
        pltpu.VMEM((KK * IW, FF), jnp.float32),
        pltpu.VMEM((ZR, FF), jnp.float32),
        pltpu.VMEM_SHARED((NP, FF), jnp.float32),
        pltpu.SemaphoreType.DMA,
        pltpu.SemaphoreType.DMA,
        pltpu.SemaphoreType.DMA,
        pltpu.SemaphoreType.DMA,
    ],
)
def _agg_sc(h_hbm, src_hbm, dst_hbm, out_hbm, sidx, didx, rows0, rows1,
            zbuf, acc, gsem0, gsem1, ssem0, ssem1):
    cid = lax.axis_index("c")
    sid = lax.axis_index("s")
    wid = cid * NS + sid

    rows = (rows0, rows1)
    gsem = (gsem0, gsem1)
    ssem = (ssem0, ssem1)

    pltpu.sync_copy(src_hbm.at[pl.ds(wid * RW, RW)], sidx)
    pltpu.sync_copy(dst_hbm.at[pl.ds(wid * RW, RW)], didx)

    @pl.loop(0, ZR)
    def _(i):
        zbuf[i, :] = jnp.zeros((FF,), jnp.float32)

    pltpu.sync_copy(zbuf, acc.at[pl.ds(sid * ZR, ZR)])
    plsc.subcore_barrier()

    def fire_gathers(b, c):
        return [pltpu.async_copy(h_hbm.at[sidx.at[c * KK + j]],
                                 rows[b].at[pl.ds(j * IW, IW)], gsem[b])
                for j in range(KK)]

    def fire_scatters(b, c):
        return [pltpu.async_copy(rows[b].at[pl.ds(j * IW, IW)],
                                 acc.at[didx.at[c * KK + j]], ssem[b],
                                 add=True)
                for j in range(KK)]

    @pl.loop(0, NG)
    def _(g):
        c0 = 2 * g
        g0 = fire_gathers(0, c0)
        g1 = fire_gathers(1, c0 + 1)
        for c in g0:
            c.wait()
        s0 = fire_scatters(0, c0)
        for c in g1:
            c.wait()
        s1 = fire_scatters(1, c0 + 1)
        for c in s0:
            c.wait()
        for c in s1:
            c.wait()

    plsc.subcore_barrier()
    pltpu.sync_copy(acc.at[pl.ds(sid * ZR, ZR)],
                    out_hbm.at[cid, pl.ds(sid * ZR, ZR)])


_BN = 1000
_GRID = NN // _BN


def _tc1_body(x_ref, w1_ref, degp_ref, h1s_ref, dis_ref):
    deg = degp_ref[0, :, :] + degp_ref[1, :, :] + 1.0
    dis = lax.rsqrt(deg)
    h1 = jnp.dot(x_ref[...], w1_ref[...], preferred_element_type=jnp.float32)
    h1s_ref[...] = h1 * dis
    dis_ref[...] = dis


def _tc1(x, W1, degp):
    return pl.pallas_call(
        _tc1_body,
        grid=(_GRID,),
        in_specs=[
            pl.BlockSpec((_BN, DD), lambda i: (i, 0)),
            pl.BlockSpec((DD, FF), lambda i: (0, 0)),
            pl.BlockSpec((NC, _BN, FF), lambda i: (0, i, 0)),
        ],
        out_specs=[
            pl.BlockSpec((_BN, FF), lambda i: (i, 0)),
            pl.BlockSpec((_BN, FF), lambda i: (i, 0)),
        ],
        out_shape=[
            jax.ShapeDtypeStruct((NN, FF), jnp.float32),
            jax.ShapeDtypeStruct((NN, FF), jnp.float32),
        ],
    )(x, W1, degp)


def _tc2_body(aggp_ref, h1s_ref, dis_ref, b1_ref, w2_ref, h2s_ref):
    agg = aggp_ref[0, :, :] + aggp_ref[1, :, :] + h1s_ref[...]
    z = jnp.maximum(dis_ref[...] * agg + b1_ref[...], 0.0)
    h2 = jnp.dot(z, w2_ref[...], preferred_element_type=jnp.float32)
    h2s_ref[...] = h2 * dis_ref[...]


def _tc2(aggp, h1s, dis, b1, W2p):
    return pl.pallas_call(
        _tc2_body,
        grid=(_GRID,),
        in_specs=[
            pl.BlockSpec((NC, _BN, FF), lambda i: (0, i, 0)),
            pl.BlockSpec((_BN, FF), lambda i: (i, 0)),
            pl.BlockSpec((_BN, FF), lambda i: (i, 0)),
            pl.BlockSpec((1, FF), lambda i: (0, 0)),
            pl.BlockSpec((FF, FF), lambda i: (0, 0)),
        ],
        out_specs=pl.BlockSpec((_BN, FF), lambda i: (i, 0)),
        out_shape=jax.ShapeDtypeStruct((NN, FF), jnp.float32),
    )(aggp, h1s, dis, b1, W2p)


def _tc3_body(aggp_ref, h2s_ref, dis_ref, b2_ref, out_ref):
    agg = aggp_ref[0, :, :] + aggp_ref[1, :, :] + h2s_ref[...]
    z = dis_ref[...] * agg + b2_ref[...]
    col = lax.broadcasted_iota(jnp.int32, (_BN, FF), 1)
    zm = jnp.where(col < CC, z, -jnp.inf)
    m = jnp.max(zm, axis=1, keepdims=True)
    lse = jnp.log(jnp.sum(jnp.exp(zm - m), axis=1, keepdims=True)) + m
    out_ref[...] = zm - lse


def _tc3(aggp, h2s, dis, b2):
    return pl.pallas_call(
        _tc3_body,
        grid=(_GRID,),
        in_specs=[
            pl.BlockSpec((NC, _BN, FF), lambda i: (0, i, 0)),
            pl.BlockSpec((_BN, FF), lambda i: (i, 0)),
            pl.BlockSpec((_BN, FF), lambda i: (i, 0)),
            pl.BlockSpec((1, FF), lambda i: (0, 0)),
        ],
        out_specs=pl.BlockSpec((_BN, FF), lambda i: (i, 0)),
        out_shape=jax.ShapeDtypeStruct((NN, FF), jnp.float32),
    )(aggp, h2s, dis, b2)


def kernel(x, edge_index, W1, b1, W2, b2):
    pad_n = EP - EE
    src_pad = jnp.zeros((pad_n,), jnp.int32)
    dst_pad = NN + (jnp.arange(pad_n, dtype=jnp.int32) % NS)
    src2d = jnp.concatenate([edge_index[0], src_pad]).reshape(RI, IW)
    dst2d = jnp.concatenate([edge_index[1], dst_pad]).reshape(RI, IW)

    W2p = jnp.zeros((FF, FF), jnp.float32).at[:, :CC].set(W2)
    b1r = b1.reshape(1, FF)
    b2r = jnp.zeros((1, FF), jnp.float32).at[0, :CC].set(b2)

    degp = _deg_sc(dst2d)
    h1s, dis = _tc1(x, W1, degp)
    agg1p = _agg_sc(h1s, src2d, dst2d)
    h2s = _tc2(agg1p, h1s, dis, b1r, W2p)
    agg2p = _agg_sc(h2s, src2d, dst2d)
    out16 = _tc3(agg2p, h2s, dis, b2r)
    return out16[:, :CC]


# speedup vs baseline: 39.5304x; 1.0201x over previous
import functools

import jax
import jax.numpy as jnp
from jax import lax
from jax.experimental import pallas as pl
from jax.experimental.pallas import tpu as pltpu
from jax.experimental.pallas import tpu_sc as plsc

NN = 10000
EE = 320000
DD = 128
FF = 16
CC = 7

NC = 2
NS = 16
NW = NC * NS

IW = 128
KK = 8
N_OUTER = 10
EP = NW * N_OUTER * KK * IW
RI = EP // IW

NP = 10112
ZR = NP // NS

_mesh = plsc.VectorSubcoreMesh(core_axis_name="c", subcore_axis_name="s")
_sc_params = pltpu.CompilerParams(use_tc_tiling_on_sc=False)


RW = N_OUTER * KK


@functools.partial(
    pl.kernel,
    out_type=jax.ShapeDtypeStruct((NC, NP, FF), jnp.float32),
    mesh=_mesh,
    compiler_params=_sc_params,
    scratch_types=[
        pltpu.VMEM((RW, IW), jnp.int32),
        pltpu.VMEM((IW, FF), jnp.float32),
        pltpu.VMEM((ZR, FF), jnp.float32),
        pltpu.VMEM_SHARED((NP, FF), jnp.float32),
        pltpu.SemaphoreType.DMA,
    ],
)
def _deg_sc(dst_hbm, out_hbm, didx, ones, zbuf, acc, ssem):
    cid = lax.axis_index("c")
    sid = lax.axis_index("s")
    wid = cid * NS + sid

    pltpu.sync_copy(dst_hbm.at[pl.ds(wid * RW, RW)], didx)

    @pl.loop(0, ZR)
    def _(i):
        zbuf[i, :] = jnp.zeros((FF,), jnp.float32)

    @pl.loop(0, IW)
    def _(i):
        ones[i, :] = jnp.ones((FF,), jnp.float32)

    pltpu.sync_copy(zbuf, acc.at[pl.ds(sid * ZR, ZR)])
    plsc.subcore_barrier()

    @pl.loop(0, N_OUTER // 2)
    def _(it):
        copies = []
        for j in range(2 * KK):
            copies.append(pltpu.async_copy(
                ones, acc.at[didx.at[it * 2 * KK + j]], ssem, add=True))
        for c in copies:
            c.wait()

    plsc.subcore_barrier()
    pltpu.sync_copy(acc.at[pl.ds(sid * ZR, ZR)],
                    out_hbm.at[cid, pl.ds(sid * ZR, ZR)])


NG = N_OUTER // 2


@functools.partial(
    pl.kernel,
    out_type=jax.ShapeDtypeStruct((NC, NP, FF), jnp.float32),
    mesh=_mesh,
    compiler_params=_sc_params,
    scratch_types=[
        pltpu.VMEM((RW, IW), jnp.int32),
        pltpu.VMEM((RW, IW), jnp.int32),
        pltpu.VMEM((KK * IW, FF), jnp.float32),
        pltpu.VMEM((KK * IW, FF), jnp.float32),
        pltpu.VMEM((ZR, FF), jnp.float32),
        pltpu.VMEM_SHARED((NP, FF), jnp.float32),
        pltpu.SemaphoreType.DMA,
        pltpu.SemaphoreType.DMA,
        pltpu.SemaphoreType.DMA,
        pltpu.SemaphoreType.DMA,
    ],
)
def _agg_sc(h_hbm, src_hbm, dst_hbm, out_hbm, sidx, didx, rows0, rows1,
            zbuf, acc, gsem0, gsem1, ssem0, ssem1):
    cid = lax.axis_index("c")
    sid = lax.axis_index("s")
    wid = cid * NS + sid

    rows = (rows0, rows1)
    gsem = (gsem0, gsem1)
    ssem = (ssem0, ssem1)

    pltpu.sync_copy(src_hbm.at[pl.ds(wid * RW, RW)], sidx)
    pltpu.sync_copy(dst_hbm.at[pl.ds(wid * RW, RW)], didx)

    @pl.loop(0, ZR)
    def _(i):
        zbuf[i, :] = jnp.zeros((FF,), jnp.float32)

    pltpu.sync_copy(zbuf, acc.at[pl.ds(sid * ZR, ZR)])
    plsc.subcore_barrier()

    def fire_gathers(b, c):
        return [pltpu.async_copy(h_hbm.at[sidx.at[c * KK + j]],
                                 rows[b].at[pl.ds(j * IW, IW)], gsem[b])
                for j in range(KK)]

    def fire_scatters(b, c):
        return [pltpu.async_copy(rows[b].at[pl.ds(j * IW, IW)],
                                 acc.at[didx.at[c * KK + j]], ssem[b],
                                 add=True)
                for j in range(KK)]

    @pl.loop(0, NG)
    def _(g):
        c0 = 2 * g
        g0 = fire_gathers(0, c0)
        g1 = fire_gathers(1, c0 + 1)
        for c in g0:
            c.wait()
        s0 = fire_scatters(0, c0)
        for c in g1:
            c.wait()
        s1 = fire_scatters(1, c0 + 1)
        for c in s0:
            c.wait()
        for c in s1:
            c.wait()

    plsc.subcore_barrier()
    pltpu.sync_copy(acc.at[pl.ds(sid * ZR, ZR)],
                    out_hbm.at[cid, pl.ds(sid * ZR, ZR)])


_BN = NN
_GRID = NN // _BN


def _tc0_body(x_ref, w1_ref, h1_ref):
    h1_ref[...] = jnp.dot(x_ref[...], w1_ref[...],
                          preferred_element_type=jnp.float32)


def _tc0(x, W1):
    return pl.pallas_call(
        _tc0_body,
        grid=(_GRID,),
        in_specs=[
            pl.BlockSpec((_BN, DD), lambda i: (i, 0)),
            pl.BlockSpec((DD, FF), lambda i: (0, 0)),
        ],
        out_specs=pl.BlockSpec((_BN, FF), lambda i: (i, 0)),
        out_shape=jax.ShapeDtypeStruct((NN, FF), jnp.float32),
    )(x, W1)


def _tc1_body(h1_ref, degp_ref, h1s_ref, dis_ref):
    deg = degp_ref[0, :, :] + degp_ref[1, :, :] + 1.0
    dis = lax.rsqrt(deg)
    h1s_ref[...] = h1_ref[...] * dis
    dis_ref[...] = dis


def _tc1(h1, degp):
    return pl.pallas_call(
        _tc1_body,
        grid=(_GRID,),
        in_specs=[
            pl.BlockSpec((_BN, FF), lambda i: (i, 0)),
            pl.BlockSpec((NC, _BN, FF), lambda i: (0, i, 0)),
        ],
        out_specs=[
            pl.BlockSpec((_BN, FF), lambda i: (i, 0)),
            pl.BlockSpec((_BN, FF), lambda i: (i, 0)),
        ],
        out_shape=[
            jax.ShapeDtypeStruct((NN, FF), jnp.float32),
            jax.ShapeDtypeStruct((NN, FF), jnp.float32),
        ],
    )(h1, degp)


def _tc2_body(aggp_ref, h1s_ref, dis_ref, b1_ref, w2_ref, h2s_ref):
    agg = aggp_ref[0, :, :] + aggp_ref[1, :, :] + h1s_ref[...]
    z = jnp.maximum(dis_ref[...] * agg + b1_ref[...], 0.0)
    h2 = jnp.dot(z, w2_ref[...], preferred_element_type=jnp.float32)
    h2s_ref[...] = h2 * dis_ref[...]


def _tc2(aggp, h1s, dis, b1, W2p):
    return pl.pallas_call(
        _tc2_body,
        grid=(_GRID,),
        in_specs=[
            pl.BlockSpec((NC, _BN, FF), lambda i: (0, i, 0)),
            pl.BlockSpec((_BN, FF), lambda i: (i, 0)),
            pl.BlockSpec((_BN, FF), lambda i: (i, 0)),
            pl.BlockSpec((1, FF), lambda i: (0, 0)),
            pl.BlockSpec((FF, FF), lambda i: (0, 0)),
        ],
        out_specs=pl.BlockSpec((_BN, FF), lambda i: (i, 0)),
        out_shape=jax.ShapeDtypeStruct((NN, FF), jnp.float32),
    )(aggp, h1s, dis, b1, W2p)


def _tc3_body(aggp_ref, h2s_ref, dis_ref, b2_ref, out_ref):
    agg = aggp_ref[0, :, :] + aggp_ref[1, :, :] + h2s_ref[...]
    z = dis_ref[...] * agg + b2_ref[...]
    col = lax.broadcasted_iota(jnp.int32, (_BN, FF), 1)
    zm = jnp.where(col < CC, z, -jnp.inf)
    m = jnp.max(zm, axis=1, keepdims=True)
    lse = jnp.log(jnp.sum(jnp.exp(zm - m), axis=1, keepdims=True)) + m
    out_ref[...] = zm - lse


def _tc3(aggp, h2s, dis, b2):
    return pl.pallas_call(
        _tc3_body,
        grid=(_GRID,),
        in_specs=[
            pl.BlockSpec((NC, _BN, FF), lambda i: (0, i, 0)),
            pl.BlockSpec((_BN, FF), lambda i: (i, 0)),
            pl.BlockSpec((_BN, FF), lambda i: (i, 0)),
            pl.BlockSpec((1, FF), lambda i: (0, 0)),
        ],
        out_specs=pl.BlockSpec((_BN, FF), lambda i: (i, 0)),
        out_shape=jax.ShapeDtypeStruct((NN, FF), jnp.float32),
    )(aggp, h2s, dis, b2)


def kernel(x, edge_index, W1, b1, W2, b2):
    pad_n = EP - EE
    src_pad = jnp.zeros((pad_n,), jnp.int32)
    dst_pad = NN + (jnp.arange(pad_n, dtype=jnp.int32) % NS)
    src2d = jnp.concatenate([edge_index[0], src_pad]).reshape(RI, IW)
    dst2d = jnp.concatenate([edge_index[1], dst_pad]).reshape(RI, IW)

    W2p = jnp.zeros((FF, FF), jnp.float32).at[:, :CC].set(W2)
    b1r = b1.reshape(1, FF)
    b2r = jnp.zeros((1, FF), jnp.float32).at[0, :CC].set(b2)

    degp = _deg_sc(dst2d)
    h1 = _tc0(x, W1)
    h1s, dis = _tc1(h1, degp)
    agg1p = _agg_sc(h1s, src2d, dst2d)
    h2s = _tc2(agg1p, h1s, dis, b1r, W2p)
    agg2p = _agg_sc(h2s, src2d, dst2d)
    out16 = _tc3(agg2p, h2s, dis, b2r)
    return out16[:, :CC]


# speedup vs baseline: 42.0798x; 1.0645x over previous
import functools

import jax
import jax.numpy as jnp
from jax import lax
from jax.experimental import pallas as pl
from jax.experimental.pallas import tpu as pltpu
from jax.experimental.pallas import tpu_sc as plsc

NN = 10000
EE = 320000
DD = 128
FF = 16
CC = 7

NC = 2
NS = 16
NW = NC * NS

IW = 128
KK = 8
N_OUTER = 10
EP = NW * N_OUTER * KK * IW
RI = EP // IW
RI_PAD = 2624

NP = 10112
ZR = NP // NS

_mesh = plsc.VectorSubcoreMesh(core_axis_name="c", subcore_axis_name="s")
_sc_params = pltpu.CompilerParams(use_tc_tiling_on_sc=False)


RW = N_OUTER * KK


@functools.partial(
    pl.kernel,
    out_type=jax.ShapeDtypeStruct((NC, NP, FF), jnp.float32),
    mesh=_mesh,
    compiler_params=_sc_params,
    scratch_types=[
        pltpu.VMEM((RW, IW), jnp.int32),
        pltpu.VMEM((IW, FF), jnp.float32),
        pltpu.VMEM((ZR, FF), jnp.float32),
        pltpu.VMEM_SHARED((NP, FF), jnp.float32),
        pltpu.SemaphoreType.DMA,
    ],
)
def _deg_sc(dst_hbm, out_hbm, didx, ones, zbuf, acc, ssem):
    cid = lax.axis_index("c")
    sid = lax.axis_index("s")
    wid = cid * NS + sid

    pltpu.sync_copy(dst_hbm.at[pl.ds(wid * RW, RW)], didx)

    @pl.loop(0, ZR)
    def _(i):
        zbuf[i, :] = jnp.zeros((FF,), jnp.float32)

    @pl.loop(0, IW)
    def _(i):
        ones[i, :] = jnp.ones((FF,), jnp.float32)

    pltpu.sync_copy(zbuf, acc.at[pl.ds(sid * ZR, ZR)])
    plsc.subcore_barrier()

    @pl.loop(0, N_OUTER // 2)
    def _(it):
        copies = []
        for j in range(2 * KK):
            copies.append(pltpu.async_copy(
                ones, acc.at[didx.at[it * 2 * KK + j]], ssem, add=True))
        for c in copies:
            c.wait()

    plsc.subcore_barrier()
    pltpu.sync_copy(acc.at[pl.ds(sid * ZR, ZR)],
                    out_hbm.at[cid, pl.ds(sid * ZR, ZR)])


NG = N_OUTER // 2

CH_FAST = 14
CH_SLOW = 6
FAST_CID = 0


@functools.partial(
    pl.kernel,
    out_type=jax.ShapeDtypeStruct((NC, NP, FF), jnp.float32),
    mesh=_mesh,
    compiler_params=_sc_params,
    scratch_types=[
        pltpu.VMEM((CH_FAST * KK, IW), jnp.int32),
        pltpu.VMEM((CH_FAST * KK, IW), jnp.int32),
        pltpu.VMEM((KK * IW, FF), jnp.float32),
        pltpu.VMEM((KK * IW, FF), jnp.float32),
        pltpu.VMEM((ZR, FF), jnp.float32),
        pltpu.VMEM_SHARED((NP, FF), jnp.float32),
        pltpu.SemaphoreType.DMA,
        pltpu.SemaphoreType.DMA,
        pltpu.SemaphoreType.DMA,
        pltpu.SemaphoreType.DMA,
    ],
)
def _agg_sc(h_hbm, src_hbm, dst_hbm, out_hbm, sidx, didx, rows0, rows1,
            zbuf, acc, gsem0, gsem1, ssem0, ssem1):
    cid = lax.axis_index("c")
    sid = lax.axis_index("s")
    wid = cid * NS + sid

    rows = (rows0, rows1)
    gsem = (gsem0, gsem1)
    ssem = (ssem0, ssem1)

    fast = cid == FAST_CID
    base = jnp.where(fast, sid * (CH_FAST * KK),
                     NS * CH_FAST * KK + sid * (CH_SLOW * KK))
    ng = jnp.where(fast, CH_FAST // 2, CH_SLOW // 2)
    pltpu.sync_copy(src_hbm.at[pl.ds(base, CH_FAST * KK)], sidx)
    pltpu.sync_copy(dst_hbm.at[pl.ds(base, CH_FAST * KK)], didx)

    @pl.loop(0, ZR)
    def _(i):
        zbuf[i, :] = jnp.zeros((FF,), jnp.float32)

    pltpu.sync_copy(zbuf, acc.at[pl.ds(sid * ZR, ZR)])
    plsc.subcore_barrier()

    def fire_gathers(b, c):
        return [pltpu.async_copy(h_hbm.at[sidx.at[c * KK + j]],
                                 rows[b].at[pl.ds(j * IW, IW)], gsem[b])
                for j in range(KK)]

    def fire_scatters(b, c):
        return [pltpu.async_copy(rows[b].at[pl.ds(j * IW, IW)],
                                 acc.at[didx.at[c * KK + j]], ssem[b],
                                 add=True)
                for j in range(KK)]

    @pl.loop(0, CH_FAST // 2)
    def _(g):
        @pl.when(g < ng)
        def _():
            c0 = 2 * g
            g0 = fire_gathers(0, c0)
            g1 = fire_gathers(1, c0 + 1)
            for c in g0:
                c.wait()
            s0 = fire_scatters(0, c0)
            for c in g1:
                c.wait()
            s1 = fire_scatters(1, c0 + 1)
            for c in s0:
                c.wait()
            for c in s1:
                c.wait()

    plsc.subcore_barrier()
    pltpu.sync_copy(acc.at[pl.ds(sid * ZR, ZR)],
                    out_hbm.at[cid, pl.ds(sid * ZR, ZR)])


_BN = NN
_GRID = NN // _BN


def _tc0_body(x_ref, w1_ref, h1_ref):
    h1_ref[...] = jnp.dot(x_ref[...], w1_ref[...],
                          preferred_element_type=jnp.float32)


def _tc0(x, W1):
    return pl.pallas_call(
        _tc0_body,
        grid=(_GRID,),
        in_specs=[
            pl.BlockSpec((_BN, DD), lambda i: (i, 0)),
            pl.BlockSpec((DD, FF), lambda i: (0, 0)),
        ],
        out_specs=pl.BlockSpec((_BN, FF), lambda i: (i, 0)),
        out_shape=jax.ShapeDtypeStruct((NN, FF), jnp.float32),
    )(x, W1)


def _tc1_body(h1_ref, degp_ref, h1s_ref, dis_ref):
    deg = degp_ref[0, :, :] + degp_ref[1, :, :] + 1.0
    dis = lax.rsqrt(deg)
    h1s_ref[...] = h1_ref[...] * dis
    dis_ref[...] = dis


def _tc1(h1, degp):
    return pl.pallas_call(
        _tc1_body,
        grid=(_GRID,),
        in_specs=[
            pl.BlockSpec((_BN, FF), lambda i: (i, 0)),
            pl.BlockSpec((NC, _BN, FF), lambda i: (0, i, 0)),
        ],
        out_specs=[
            pl.BlockSpec((_BN, FF), lambda i: (i, 0)),
            pl.BlockSpec((_BN, FF), lambda i: (i, 0)),
        ],
        out_shape=[
            jax.ShapeDtypeStruct((NN, FF), jnp.float32),
            jax.ShapeDtypeStruct((NN, FF), jnp.float32),
        ],
    )(h1, degp)


def _tc2_body(aggp_ref, h1s_ref, dis_ref, b1_ref, w2_ref, h2s_ref):
    agg = aggp_ref[0, :, :] + aggp_ref[1, :, :] + h1s_ref[...]
    z = jnp.maximum(dis_ref[...] * agg + b1_ref[...], 0.0)
    h2 = jnp.dot(z, w2_ref[...], preferred_element_type=jnp.float32)
    h2s_ref[...] = h2 * dis_ref[...]


def _tc2(aggp, h1s, dis, b1, W2p):
    return pl.pallas_call(
        _tc2_body,
        grid=(_GRID,),
        in_specs=[
            pl.BlockSpec((NC, _BN, FF), lambda i: (0, i, 0)),
            pl.BlockSpec((_BN, FF), lambda i: (i, 0)),
            pl.BlockSpec((_BN, FF), lambda i: (i, 0)),
            pl.BlockSpec((1, FF), lambda i: (0, 0)),
            pl.BlockSpec((FF, FF), lambda i: (0, 0)),
        ],
        out_specs=pl.BlockSpec((_BN, FF), lambda i: (i, 0)),
        out_shape=jax.ShapeDtypeStruct((NN, FF), jnp.float32),
    )(aggp, h1s, dis, b1, W2p)


def _tc3_body(aggp_ref, h2s_ref, dis_ref, b2_ref, out_ref):
    agg = aggp_ref[0, :, :] + aggp_ref[1, :, :] + h2s_ref[...]
    z = dis_ref[...] * agg + b2_ref[...]
    col = lax.broadcasted_iota(jnp.int32, (_BN, FF), 1)
    zm = jnp.where(col < CC, z, -jnp.inf)
    m = jnp.max(zm, axis=1, keepdims=True)
    lse = jnp.log(jnp.sum(jnp.exp(zm - m), axis=1, keepdims=True)) + m
    out_ref[...] = zm - lse


def _tc3(aggp, h2s, dis, b2):
    return pl.pallas_call(
        _tc3_body,
        grid=(_GRID,),
        in_specs=[
            pl.BlockSpec((NC, _BN, FF), lambda i: (0, i, 0)),
            pl.BlockSpec((_BN, FF), lambda i: (i, 0)),
            pl.BlockSpec((_BN, FF), lambda i: (i, 0)),
            pl.BlockSpec((1, FF), lambda i: (0, 0)),
        ],
        out_specs=pl.BlockSpec((_BN, FF), lambda i: (i, 0)),
        out_shape=jax.ShapeDtypeStruct((NN, FF), jnp.float32),
    )(aggp, h2s, dis, b2)


def kernel(x, edge_index, W1, b1, W2, b2):
    pad_n = RI_PAD * IW - EE
    src_pad = jnp.zeros((pad_n,), jnp.int32)
    dst_pad = NN + (jnp.arange(pad_n, dtype=jnp.int32) % NS)
    src2d = jnp.concatenate([edge_index[0], src_pad]).reshape(RI_PAD, IW)
    dst2d = jnp.concatenate([edge_index[1], dst_pad]).reshape(RI_PAD, IW)

    W2p = jnp.zeros((FF, FF), jnp.float32).at[:, :CC].set(W2)
    b1r = b1.reshape(1, FF)
    b2r = jnp.zeros((1, FF), jnp.float32).at[0, :CC].set(b2)

    degp = _deg_sc(dst2d)
    h1 = _tc0(x, W1)
    h1s, dis = _tc1(h1, degp)
    agg1p = _agg_sc(h1s, src2d, dst2d)
    h2s = _tc2(agg1p, h1s, dis, b1r, W2p)
    agg2p = _agg_sc(h2s, src2d, dst2d)
    out16 = _tc3(agg2p, h2s, dis, b2r)
    return out16[:, :CC]
